# Initial kernel scaffold; baseline (speedup 1.0000x reference)
import jax
import jax.numpy as jnp
from jax.experimental import pallas as pl


def kernel(input, target):
    raise NotImplementedError("write your pallas kernel here")


# speedup vs baseline: 899.1209x; 899.1209x over previous
import jax
import jax.numpy as jnp
from jax.experimental import pallas as pl

_N = 4194304
_COLS = 1024
_ROWS = _N // _COLS
_BLOCK_ROWS = 512
_GRID = _ROWS // _BLOCK_ROWS


def _mse_body(i_ref, t_ref, o_ref):
    @pl.when(pl.program_id(0) == 0)
    def _init():
        o_ref[...] = jnp.zeros_like(o_ref)

    d = t_ref[...] - i_ref[...]
    o_ref[...] += (jnp.sum(d * d) * (1.0 / _N)).reshape(1, 1)


def kernel(input, target):
    inp2 = input.reshape(_ROWS, _COLS)
    tgt2 = target.reshape(_ROWS, _COLS)
    out = pl.pallas_call(
        _mse_body,
        grid=(_GRID,),
        in_specs=[
            pl.BlockSpec((_BLOCK_ROWS, _COLS), lambda i: (i, 0)),
            pl.BlockSpec((_BLOCK_ROWS, _COLS), lambda i: (i, 0)),
        ],
        out_specs=pl.BlockSpec((1, 1), lambda i: (0, 0)),
        out_shape=jax.ShapeDtypeStruct((1, 1), jnp.float32),
    )(inp2, tgt2)
    return out[0, 0]
